# baseline (speedup 1.0000x reference)
import functools

import jax
import jax.numpy as jnp
from jax import lax
from jax.experimental import pallas as pl
from jax.experimental.pallas import tpu as pltpu
from jax.experimental.pallas import tpu_sc as plsc

VOCAB = 1000000
WDIM = 64
PDIM = 50
ODIM = WDIM + PDIM
MAXLEN = 31
B = 16384
TOK = B * MAXLEN

NC, NS, L = 2, 16, 16
NW = NC * NS

ROWS_W = B // NW
ROWS_C = 16
CHUNKS = ROWS_W // ROWS_C
C = ROWS_C * MAXLEN
GSUB = 128


def _body(idx_hbm, dst_hbm, len_hbm, word_hbm, dt_hbm, out_hbm,
          idx_v, dst_v, len_v, lenexp, m_v, zbuf, wslab, dslab, oslab,
          sem_w, sem_d):
    wid = lax.axis_index("s") * NC + lax.axis_index("c")
    iota = lax.iota(jnp.int32, L)
    ones_f = jnp.ones((L,), jnp.float32)
    zero_f = jnp.zeros((L,), jnp.float32)
    zero_i = jnp.zeros((L,), jnp.int32)
    vml = jnp.full((L,), MAXLEN, jnp.int32)
    zbuf[:] = iota
    rtzero = zbuf[:] - iota

    @pl.loop(0, CHUNKS)
    def _chunk(c):
        rowbase = wid * ROWS_W + c * ROWS_C
        tokbase = rowbase * MAXLEN

        pltpu.sync_copy(idx_hbm.at[pl.ds(tokbase, C)], idx_v)
        pltpu.sync_copy(dst_hbm.at[pl.ds(tokbase, C)], dst_v)
        pltpu.sync_copy(len_hbm.at[pl.ds(rowbase, ROWS_C)], len_v)

        @pl.loop(0, ROWS_C)
        def _row(r):
            rv = lax.broadcast_in_dim(r.astype(jnp.int32), (L,), ()) + rtzero
            lvr = plsc.load_gather(len_v, [rv])
            lenexp[pl.ds(r * MAXLEN, L)] = lvr
            lenexp[pl.ds(r * MAXLEN + MAXLEN - L, L)] = lvr

        for g in range(C // L):
            e = jnp.full((L,), g * L, jnp.int32) + iota
            brow = lax.div(e, vml)
            pos = e - brow * vml
            lv = lenexp[pl.ds(g * L, L)]
            msk = pos < lv
            m_v[pl.ds(g * L, L)] = jnp.where(msk, ones_f, zero_f)
            dcur = dst_v[pl.ds(g * L, L)]
            dst_v[pl.ds(g * L, L)] = jnp.where(msk, dcur, zero_i)

        copies = []
        off = 0
        while off < C:
            n = min(GSUB, C - off)
            copies.append(pltpu.async_copy(
                word_hbm.at[idx_v.at[pl.ds(off, n)]],
                wslab.at[pl.ds(off, n)], sem_w))
            copies.append(pltpu.async_copy(
                dt_hbm.at[dst_v.at[pl.ds(off, n)]],
                dslab.at[pl.ds(off, n)], sem_d))
            off += n
        for cp in copies:
            cp.wait()

        @pl.loop(0, C)
        def _tok(t):
            tv = lax.broadcast_in_dim(t.astype(jnp.int32), (L,), ()) + rtzero
            m16 = plsc.load_gather(m_v, [tv])
            for i in range(WDIM // L):
                oslab[t, pl.ds(i * L, L)] = wslab[t, pl.ds(i * L, L)] * m16
            for o in (0, 16, 32, PDIM - L):
                oslab[t, pl.ds(WDIM + o, L)] = dslab[t, pl.ds(o, L)]

        pltpu.sync_copy(oslab, out_hbm.at[pl.ds(tokbase, C), :])


@jax.jit
def _run(idx_f, dst_f, length, word_table, dt0):
    mesh = plsc.VectorSubcoreMesh(core_axis_name="c", subcore_axis_name="s")
    return pl.kernel(
        _body,
        out_type=jax.ShapeDtypeStruct((TOK, ODIM), jnp.float32),
        mesh=mesh,
        compiler_params=pltpu.CompilerParams(
            needs_layout_passes=False, use_tc_tiling_on_sc=False),
        scratch_types=[
            pltpu.VMEM((C,), jnp.int32),
            pltpu.VMEM((C,), jnp.int32),
            pltpu.VMEM((ROWS_C,), jnp.int32),
            pltpu.VMEM((C + L,), jnp.int32),
            pltpu.VMEM((C,), jnp.float32),
            pltpu.VMEM((L,), jnp.int32),
            pltpu.VMEM((C, WDIM), jnp.float32),
            pltpu.VMEM((C, WDIM), jnp.float32),
            pltpu.VMEM((C, ODIM), jnp.float32),
            pltpu.SemaphoreType.DMA,
            pltpu.SemaphoreType.DMA,
        ],
    )(idx_f, dst_f, length, word_table, dt0)


def kernel(indices, dist, length, word_table, dist_table):
    dt0 = jnp.zeros((dist_table.shape[0], WDIM), dist_table.dtype)
    dt0 = dt0.at[:, :PDIM].set(dist_table).at[0].set(0.0)
    out = _run(indices.reshape(-1), dist.reshape(-1), length.reshape(-1),
               word_table, dt0)
    return out.reshape(B, MAXLEN, ODIM)
